# baseline (speedup 1.0000x reference)
import jax
import jax.numpy as jnp
from jax import lax
from jax.experimental import pallas as pl
from jax.experimental.pallas import tpu as pltpu
from jax.experimental.pallas import tpu_sc as plsc

_NC = 2
_NS = 16
_NW = _NC * _NS

_D = 32
_BATCH = 16384
_NPOS = 100
_W = _BATCH // _NW
_L = 16


def _gather_kernel(table_hbm, idx_hbm, out_hbm, *scratch):
    idxs = scratch[0:4]
    rows = scratch[4:8]
    valst = scratch[8:10]
    isem = scratch[10:14]
    gsem = scratch[14:18]
    wsem = scratch[18:20]
    skew = scratch[20]

    wid = lax.axis_index("s") * _NC + lax.axis_index("c")
    b0 = wid * _W

    def idx_copy(p, b):
        return pltpu.make_async_copy(
            idx_hbm.at[p, pl.ds(b0, _W)], idxs[b], isem[b])

    def gather(b):
        return pltpu.make_async_copy(table_hbm.at[idxs[b]], rows[b], gsem[b])

    tcb = wid * (_W // 128)

    def out_copy(p, s):
        return pltpu.make_async_copy(
            valst[s], out_hbm.at[p, :, pl.ds(tcb, _W // 128), :, :], wsem[s])

    iota = lax.iota(jnp.int32, _L)

    for b in range(4):
        idx_copy(b, b).start()
    for b in range(3):
        idx_copy(b, b).wait()
        gather(b).start()

    @pl.loop(0, _NPOS, step=4)
    def _quad(p0):
        for k in range(4):
            p = p0 + k
            s = k % 2

            gather(k).wait()

            @pl.when(p + 4 < _NPOS)
            def _():
                idx_copy(p + 4, k).start()

            @pl.when(p + 3 < _NPOS)
            def _():
                idx_copy(p + 3, (k + 3) % 4).wait()
                gather((k + 3) % 4).start()

            if k < 2:
                @pl.when(p0 >= 4)
                def _():
                    out_copy(p - 2, s).wait()
            else:
                out_copy(p - 2, s).wait()

            @plsc.parallel_loop(0, _W, 1, unroll=8)
            def _skew(b):
                for h in range(2):
                    skew[pl.ds(b * 33 + h * _L, _L)] = (
                        rows[k][b, pl.ds(h * _L, _L)])

            @plsc.parallel_loop(0, _W, _L, unroll=4)
            def _blk(bb):
                ridx33 = (bb + iota) * 33
                tcl = bb // 128
                cc0 = bb % 128
                for c in range(_D):
                    v = plsc.load_gather(skew, [ridx33 + c])
                    valst[s][c // 8, tcl, c % 8, pl.ds(cc0, _L)] = v

            out_copy(p, s).start()

    for s in range(2):
        out_copy(_NPOS - 2 + s, s).wait()


_ROWS = 1_000_000
_FULL = 999_936
_NCOLS = _FULL // 128
_TAIL = _ROWS - _FULL


def _convert_kernel(embt_hbm, tail_hbm, out_hbm, *scratch):
    blk = scratch[0:2]
    vals = scratch[2:4]
    lsem = scratch[4:6]
    wsem = scratch[6:8]
    tailv = scratch[8]
    tailt = scratch[9]
    bskew = scratch[10]

    wid = lax.axis_index("s") * _NC + lax.axis_index("c")
    base = wid * 244 + jnp.minimum(wid, 4)
    iota = lax.iota(jnp.int32, _L)

    def load(j, s):
        return pltpu.make_async_copy(
            embt_hbm.at[:, pl.ds((base + j) * 128, 128)], blk[s], lsem[s])

    def store(j, s):
        return pltpu.make_async_copy(
            vals[s], out_hbm.at[pl.ds((base + j) * 128 * _D, 128 * _D)],
            wsem[s])

    rconst = [(h * _L + iota) * 133 for h in range(2)]

    def transpose(s):
        @plsc.parallel_loop(0, _D, 1, unroll=4)
        def _c(c):
            for q in range(8):
                bskew[pl.ds(c * 133 + q * _L, _L)] = (
                    blk[s][c, pl.ds(q * _L, _L)])

        @plsc.parallel_loop(0, 128, 1, unroll=4)
        def _cc(cc):
            for h in range(2):
                v = plsc.load_gather(bskew, [rconst[h] + cc])
                vals[s][pl.ds(cc * _D + h * _L, _L)] = v

    for s in range(2):
        load(s, s).start()

    @pl.loop(0, 244, step=2)
    def _pairs(j0):
        for s in range(2):
            j = j0 + s
            load(j, s).wait()

            @pl.when(j0 >= 2)
            def _():
                store(j - 2, s).wait()

            transpose(s)

            @pl.when(j0 + 4 <= 244)
            def _():
                load(j + 2, s).start()

            store(j, s).start()

    for s in range(2):
        store(242 + s, s).wait()

    @pl.when(wid < 4)
    def _():
        pltpu.sync_copy(
            embt_hbm.at[:, pl.ds((base + 244) * 128, 128)], blk[0])
        transpose(0)
        pltpu.sync_copy(
            vals[0], out_hbm.at[pl.ds((base + 244) * 128 * _D, 128 * _D)])

    @pl.when(wid == 31)
    def _():
        pltpu.sync_copy(tail_hbm, tailv)

        @plsc.parallel_loop(0, _TAIL, 1, unroll=4)
        def _rr(rr):
            for h in range(2):
                v = plsc.load_gather(
                    tailv, [(h * _L + iota) * _TAIL + rr])
                tailt[pl.ds(rr * _D + h * _L, _L)] = v
        pltpu.sync_copy(tailt, out_hbm.at[pl.ds(_FULL * _D, _TAIL * _D)])


@jax.jit
def _convert(embt, tail_flat):
    mesh = plsc.VectorSubcoreMesh(
        core_axis_name="c", subcore_axis_name="s",
        num_cores=_NC, num_subcores=_NS)
    return pl.kernel(
        _convert_kernel,
        out_type=jax.ShapeDtypeStruct((_ROWS * _D,), jnp.float32),
        mesh=mesh,
        scratch_types=(
            [pltpu.VMEM((_D, 128), jnp.float32) for _ in range(2)]
            + [pltpu.VMEM((128 * _D,), jnp.float32) for _ in range(2)]
            + [pltpu.SemaphoreType.DMA for _ in range(4)]
            + [pltpu.VMEM((_TAIL * _D,), jnp.float32),
               pltpu.VMEM((_TAIL * _D,), jnp.float32),
               pltpu.VMEM((_D * 133,), jnp.float32)]
        ),
        compiler_params=pltpu.CompilerParams(
            use_tc_tiling_on_sc=True, needs_layout_passes=False),
    )(embt, tail_flat)


@jax.jit
def _lookup(embed, idx_t):
    mesh = plsc.VectorSubcoreMesh(
        core_axis_name="c", subcore_axis_name="s",
        num_cores=_NC, num_subcores=_NS)
    return pl.kernel(
        _gather_kernel,
        out_type=jax.ShapeDtypeStruct(
            (_NPOS, _D // 8, _BATCH // 128, 8, 128), jnp.float32),
        mesh=mesh,
        scratch_types=(
            [pltpu.VMEM((_W,), jnp.int32) for _ in range(4)]
            + [pltpu.VMEM((_W, _D), jnp.float32) for _ in range(4)]
            + [pltpu.VMEM((_D // 8, _W // 128, 8, 128), jnp.float32)
               for _ in range(2)]
            + [pltpu.SemaphoreType.DMA for _ in range(10)]
            + [pltpu.VMEM((_W * 33,), jnp.float32)]
        ),
        compiler_params=pltpu.CompilerParams(
            use_tc_tiling_on_sc=False, needs_layout_passes=False),
    )(embed, idx_t)


def kernel(embed, indices):
    idx_t = jnp.swapaxes(indices, 0, 1).astype(jnp.int32)
    tail_flat = embed[_FULL:].T.reshape(-1)
    table_rm = _convert(embed.T, tail_flat).reshape(_ROWS, _D)
    out5 = _lookup(table_rm, idx_t)
    out3 = out5.transpose(0, 1, 3, 2, 4).reshape(_NPOS, _D, _BATCH)
    return out3.transpose(2, 0, 1)


# speedup vs baseline: 1.0482x; 1.0482x over previous
import jax
import jax.numpy as jnp
from jax import lax
from jax.experimental import pallas as pl
from jax.experimental.pallas import tpu as pltpu
from jax.experimental.pallas import tpu_sc as plsc

_NC = 2
_NS = 16
_NW = _NC * _NS

_D = 32
_BATCH = 16384
_NPOS = 100
_W = _BATCH // _NW
_L = 16


def _gather_kernel(table_hbm, idx_hbm, out_hbm, *scratch):
    idxs = scratch[0:4]
    rows = scratch[4:8]
    valst = scratch[8:10]
    isem = scratch[10:14]
    gsem = scratch[14:18]
    wsem = scratch[18:20]
    skew = scratch[20]

    wid = lax.axis_index("s") * _NC + lax.axis_index("c")
    b0 = wid * _W

    def idx_copy(p, b):
        return pltpu.make_async_copy(
            idx_hbm.at[p, pl.ds(b0, _W)], idxs[b], isem[b])

    def gather(b):
        return pltpu.make_async_copy(table_hbm.at[idxs[b]], rows[b], gsem[b])

    tcb = wid * (_W // 128)

    def out_copy(p, s):
        return pltpu.make_async_copy(
            valst[s], out_hbm.at[p, :, pl.ds(tcb, _W // 128), :, :], wsem[s])

    iota = lax.iota(jnp.int32, _L)

    for b in range(4):
        idx_copy(b, b).start()
    for b in range(3):
        idx_copy(b, b).wait()
        gather(b).start()

    @pl.loop(0, _NPOS, step=4)
    def _quad(p0):
        for k in range(4):
            p = p0 + k
            s = k % 2

            gather(k).wait()

            @pl.when(p + 4 < _NPOS)
            def _():
                idx_copy(p + 4, k).start()

            @pl.when(p + 3 < _NPOS)
            def _():
                idx_copy(p + 3, (k + 3) % 4).wait()
                gather((k + 3) % 4).start()

            if k < 2:
                @pl.when(p0 >= 4)
                def _():
                    out_copy(p - 2, s).wait()
            else:
                out_copy(p - 2, s).wait()

            @plsc.parallel_loop(0, _W, 1, unroll=8)
            def _skew(b):
                for h in range(2):
                    skew[pl.ds(b * 33 + h * _L, _L)] = (
                        rows[k][b, pl.ds(h * _L, _L)])

            @plsc.parallel_loop(0, _W, _L, unroll=2)
            def _blk(bb):
                ridx33 = (bb + iota) * 33
                tcl = bb // 128
                cc0 = bb % 128
                for c in range(_D):
                    v = plsc.load_gather(skew, [ridx33 + c])
                    valst[s][c // 8, tcl, c % 8, pl.ds(cc0, _L)] = v

            out_copy(p, s).start()

    for s in range(2):
        out_copy(_NPOS - 2 + s, s).wait()


_ROWS = 1_000_000
_FULL = 999_936
_NCOLS = _FULL // 128
_TAIL = _ROWS - _FULL


def _convert_kernel(embt_hbm, tail_hbm, out_hbm, *scratch):
    blk = scratch[0:2]
    vals = scratch[2:4]
    lsem = scratch[4:6]
    wsem = scratch[6:8]
    tailv = scratch[8]
    tailt = scratch[9]
    bskew = scratch[10]

    wid = lax.axis_index("s") * _NC + lax.axis_index("c")
    base = wid * 244 + jnp.minimum(wid, 4)
    iota = lax.iota(jnp.int32, _L)

    def load(j, s):
        return pltpu.make_async_copy(
            embt_hbm.at[:, pl.ds((base + j) * 128, 128)], blk[s], lsem[s])

    def store(j, s):
        return pltpu.make_async_copy(
            vals[s], out_hbm.at[pl.ds((base + j) * 128 * _D, 128 * _D)],
            wsem[s])

    rconst = [(h * _L + iota) * 133 for h in range(2)]

    def transpose(s):
        @plsc.parallel_loop(0, _D, 1, unroll=4)
        def _c(c):
            for q in range(8):
                bskew[pl.ds(c * 133 + q * _L, _L)] = (
                    blk[s][c, pl.ds(q * _L, _L)])

        @plsc.parallel_loop(0, 128, 1, unroll=4)
        def _cc(cc):
            for h in range(2):
                v = plsc.load_gather(bskew, [rconst[h] + cc])
                vals[s][pl.ds(cc * _D + h * _L, _L)] = v

    for s in range(2):
        load(s, s).start()

    @pl.loop(0, 244, step=2)
    def _pairs(j0):
        for s in range(2):
            j = j0 + s
            load(j, s).wait()

            @pl.when(j0 >= 2)
            def _():
                store(j - 2, s).wait()

            transpose(s)

            @pl.when(j0 + 4 <= 244)
            def _():
                load(j + 2, s).start()

            store(j, s).start()

    for s in range(2):
        store(242 + s, s).wait()

    @pl.when(wid < 4)
    def _():
        pltpu.sync_copy(
            embt_hbm.at[:, pl.ds((base + 244) * 128, 128)], blk[0])
        transpose(0)
        pltpu.sync_copy(
            vals[0], out_hbm.at[pl.ds((base + 244) * 128 * _D, 128 * _D)])

    @pl.when(wid == 31)
    def _():
        pltpu.sync_copy(tail_hbm, tailv)

        @plsc.parallel_loop(0, _TAIL, 1, unroll=4)
        def _rr(rr):
            for h in range(2):
                v = plsc.load_gather(
                    tailv, [(h * _L + iota) * _TAIL + rr])
                tailt[pl.ds(rr * _D + h * _L, _L)] = v
        pltpu.sync_copy(tailt, out_hbm.at[pl.ds(_FULL * _D, _TAIL * _D)])


@jax.jit
def _convert(embt, tail_flat):
    mesh = plsc.VectorSubcoreMesh(
        core_axis_name="c", subcore_axis_name="s",
        num_cores=_NC, num_subcores=_NS)
    return pl.kernel(
        _convert_kernel,
        out_type=jax.ShapeDtypeStruct((_ROWS * _D,), jnp.float32),
        mesh=mesh,
        scratch_types=(
            [pltpu.VMEM((_D, 128), jnp.float32) for _ in range(2)]
            + [pltpu.VMEM((128 * _D,), jnp.float32) for _ in range(2)]
            + [pltpu.SemaphoreType.DMA for _ in range(4)]
            + [pltpu.VMEM((_TAIL * _D,), jnp.float32),
               pltpu.VMEM((_TAIL * _D,), jnp.float32),
               pltpu.VMEM((_D * 133,), jnp.float32)]
        ),
        compiler_params=pltpu.CompilerParams(
            use_tc_tiling_on_sc=True, needs_layout_passes=False),
    )(embt, tail_flat)


@jax.jit
def _lookup(embed, idx_t):
    mesh = plsc.VectorSubcoreMesh(
        core_axis_name="c", subcore_axis_name="s",
        num_cores=_NC, num_subcores=_NS)
    return pl.kernel(
        _gather_kernel,
        out_type=jax.ShapeDtypeStruct(
            (_NPOS, _D // 8, _BATCH // 128, 8, 128), jnp.float32),
        mesh=mesh,
        scratch_types=(
            [pltpu.VMEM((_W,), jnp.int32) for _ in range(4)]
            + [pltpu.VMEM((_W, _D), jnp.float32) for _ in range(4)]
            + [pltpu.VMEM((_D // 8, _W // 128, 8, 128), jnp.float32)
               for _ in range(2)]
            + [pltpu.SemaphoreType.DMA for _ in range(10)]
            + [pltpu.VMEM((_W * 33,), jnp.float32)]
        ),
        compiler_params=pltpu.CompilerParams(
            use_tc_tiling_on_sc=False, needs_layout_passes=False),
    )(embed, idx_t)


def kernel(embed, indices):
    idx_t = jnp.swapaxes(indices, 0, 1).astype(jnp.int32)
    tail_flat = embed[_FULL:].T.reshape(-1)
    table_rm = _convert(embed.T, tail_flat).reshape(_ROWS, _D)
    out5 = _lookup(table_rm, idx_t)
    out3 = out5.transpose(0, 1, 3, 2, 4).reshape(_NPOS, _D, _BATCH)
    return out3.transpose(2, 0, 1)
